# baseline (speedup 1.0000x reference)
import functools

import jax
import jax.numpy as jnp
from jax import lax
from jax.experimental import pallas as pl
from jax.experimental.pallas import tpu as pltpu
from jax.experimental.pallas import tpu_sc as plsc

_B = 16384
_D = 64
_NC, _NS = 2, 16
_NW = _NC * _NS
_BPW = _B // _NW
_CH = 128
_NCH = _BPW // _CH


def _body(uids, iids, ut, it, uout, iout, uidx, iidx, urows, irows, usem, isem):
    wid = lax.axis_index("s") * _NC + lax.axis_index("c")
    base = wid * _BPW
    pltpu.sync_copy(uids.at[pl.ds(base, _BPW)], uidx)
    pltpu.sync_copy(iids.at[pl.ds(base, _BPW)], iidx)
    ucp = [
        pltpu.async_copy(
            ut.at[uidx.at[pl.ds(j * _CH, _CH)]],
            urows.at[pl.ds(j * _CH, _CH)],
            usem,
        )
        for j in range(_NCH)
    ]
    icp = [
        pltpu.async_copy(
            it.at[iidx.at[pl.ds(j * _CH, _CH)]],
            irows.at[pl.ds(j * _CH, _CH)],
            isem,
        )
        for j in range(_NCH)
    ]
    for c in ucp:
        c.wait()
    pltpu.sync_copy(urows, uout.at[pl.ds(base, _BPW)])
    for c in icp:
        c.wait()
    pltpu.sync_copy(irows, iout.at[pl.ds(base, _BPW)])


_gather = functools.partial(
    pl.kernel,
    mesh=plsc.VectorSubcoreMesh(core_axis_name="c", subcore_axis_name="s"),
    compiler_params=pltpu.CompilerParams(use_tc_tiling_on_sc=False),
    out_type=(
        jax.ShapeDtypeStruct((_B, _D), jnp.float32),
        jax.ShapeDtypeStruct((_B, _D), jnp.float32),
    ),
    scratch_types=[
        pltpu.VMEM((_BPW,), jnp.int32),
        pltpu.VMEM((_BPW,), jnp.int32),
        pltpu.VMEM((_BPW, _D), jnp.float32),
        pltpu.VMEM((_BPW, _D), jnp.float32),
        pltpu.SemaphoreType.DMA,
        pltpu.SemaphoreType.DMA,
    ],
)(_body)


def kernel(user_ids, item_ids, user_table, item_table):
    return _gather(
        user_ids.astype(jnp.int32),
        item_ids.astype(jnp.int32),
        user_table,
        item_table,
    )
